# Initial kernel scaffold; baseline (speedup 1.0000x reference)
import functools

import jax
import jax.numpy as jnp
from jax import lax
from jax.experimental import pallas as pl
from jax.experimental.pallas import tpu as pltpu
from jax.experimental.pallas import tpu_sc as plsc

BN = 256


def _vq_dist_body(x_ref, w_ref, idx_ref, losssum_ref):
    nt = pl.program_id(0)
    x = x_ref[...]
    w = w_ref[...]
    k = w.shape[0]
    x2 = jnp.sum(x * x, axis=1, keepdims=True)
    z2 = jnp.sum(w * w, axis=1)
    s = lax.dot_general(x, w, (((1,), (1,)), ((), ())),
                        preferred_element_type=jnp.float32)
    d = (x2 + z2[None, :]) - 2.0 * s
    dmin = jnp.min(d, axis=1, keepdims=True)
    iota = lax.broadcasted_iota(jnp.int32, d.shape, 1)
    idx = jnp.min(jnp.where(d == dmin, iota, k), axis=1)
    idx_ref[...] = idx

    @pl.when(nt == 0)
    def _():
        losssum_ref[0, 0] = 0.0

    losssum_ref[0, 0] += jnp.sum(dmin)


def _vq_distances(x, embed_weight):
    n, d = x.shape
    k, _ = embed_weight.shape
    grid = (n // BN,)
    return pl.pallas_call(
        _vq_dist_body,
        grid=grid,
        in_specs=[
            pl.BlockSpec((BN, d), lambda i: (i, 0)),
            pl.BlockSpec((k, d), lambda i: (0, 0)),
        ],
        out_specs=[
            pl.BlockSpec((BN,), lambda i: (i,)),
            pl.BlockSpec((1, 1), lambda i: (0, 0)),
        ],
        out_shape=[
            jax.ShapeDtypeStruct((n,), jnp.int32),
            jax.ShapeDtypeStruct((1, 1), jnp.float32),
        ],
    )(x, embed_weight)


def _make_sc_gather(v, d, b):
    info = plsc.get_sparse_core_info()
    nw = info.num_cores * info.num_subcores
    b_per_w = b // nw
    chunk = 64
    n_chunks = b_per_w // chunk
    mesh = plsc.VectorSubcoreMesh(core_axis_name="c", subcore_axis_name="s")

    @functools.partial(
        pl.kernel,
        mesh=mesh,
        out_type=jax.ShapeDtypeStruct((b, d), jnp.float32),
        scratch_types=[
            pltpu.VMEM((b_per_w,), jnp.int32),
            pltpu.VMEM((chunk, d), jnp.float32),
            pltpu.SemaphoreType.DMA,
        ],
    )
    def gather_kernel(table_hbm, idx_hbm, out_hbm, idx_v, rows_v, sem):
        wid = lax.axis_index("s") * info.num_cores + lax.axis_index("c")
        base = wid * b_per_w
        pltpu.sync_copy(idx_hbm.at[pl.ds(base, b_per_w)], idx_v)

        def chunk_body(c, carry):
            pltpu.async_copy(
                table_hbm.at[idx_v.at[pl.ds(c * chunk, chunk)]], rows_v, sem
            ).wait()
            pltpu.sync_copy(rows_v, out_hbm.at[pl.ds(base + c * chunk, chunk)])
            return carry

        lax.fori_loop(0, n_chunks, chunk_body, 0)

    return gather_kernel


def kernel(x, embed_weight):
    n, dim = x.shape
    k, _ = embed_weight.shape
    idx, losssum = _vq_distances(x, embed_weight)
    vq_out = _make_sc_gather(k, dim, n)(embed_weight, idx)
    loss = losssum[0, 0] * (1.25 / (n * dim))
    return (vq_out, loss)


# speedup vs baseline: 1.1575x; 1.1575x over previous
import functools

import jax
import jax.numpy as jnp
from jax import lax
from jax.experimental import pallas as pl
from jax.experimental.pallas import tpu as pltpu
from jax.experimental.pallas import tpu_sc as plsc

BN = 256


def _vq_dist_body(x_ref, w_ref, idx_ref, losssum_ref):
    nt = pl.program_id(0)
    x = x_ref[...]
    w = w_ref[...]
    k = w.shape[0]
    x2 = jnp.sum(x * x, axis=1, keepdims=True)
    z2 = jnp.sum(w * w, axis=1)
    s = lax.dot_general(x, w, (((1,), (1,)), ((), ())),
                        preferred_element_type=jnp.float32)
    d = (x2 + z2[None, :]) - 2.0 * s
    dmin = jnp.min(d, axis=1, keepdims=True)
    iota = lax.broadcasted_iota(jnp.int32, d.shape, 1)
    idx = jnp.min(jnp.where(d == dmin, iota, k), axis=1)
    idx_ref[...] = idx

    @pl.when(nt == 0)
    def _():
        losssum_ref[0, 0] = 0.0

    losssum_ref[0, 0] += jnp.sum(dmin)


def _vq_distances(x, embed_weight):
    n, d = x.shape
    k, _ = embed_weight.shape
    grid = (n // BN,)
    return pl.pallas_call(
        _vq_dist_body,
        grid=grid,
        in_specs=[
            pl.BlockSpec((BN, d), lambda i: (i, 0)),
            pl.BlockSpec((k, d), lambda i: (0, 0)),
        ],
        out_specs=[
            pl.BlockSpec((BN,), lambda i: (i,)),
            pl.BlockSpec(memory_space=pltpu.SMEM),
        ],
        out_shape=[
            jax.ShapeDtypeStruct((n,), jnp.int32),
            jax.ShapeDtypeStruct((1, 1), jnp.float32),
        ],
    )(x, embed_weight)


def _make_sc_gather(v, d, b):
    info = plsc.get_sparse_core_info()
    nw = info.num_cores * info.num_subcores
    b_per_w = b // nw
    chunk = 64
    n_chunks = b_per_w // chunk
    mesh = plsc.VectorSubcoreMesh(core_axis_name="c", subcore_axis_name="s")

    @functools.partial(
        pl.kernel,
        mesh=mesh,
        out_type=jax.ShapeDtypeStruct((b, d), jnp.float32),
        scratch_types=[
            pltpu.VMEM((b_per_w,), jnp.int32),
            pltpu.VMEM((chunk, d), jnp.float32),
            pltpu.SemaphoreType.DMA,
        ],
    )
    def gather_kernel(table_hbm, idx_hbm, out_hbm, idx_v, rows_v, sem):
        wid = lax.axis_index("s") * info.num_cores + lax.axis_index("c")
        base = wid * b_per_w
        pltpu.sync_copy(idx_hbm.at[pl.ds(base, b_per_w)], idx_v)

        def chunk_body(c, carry):
            pltpu.async_copy(
                table_hbm.at[idx_v.at[pl.ds(c * chunk, chunk)]], rows_v, sem
            ).wait()
            pltpu.sync_copy(rows_v, out_hbm.at[pl.ds(base + c * chunk, chunk)])
            return carry

        lax.fori_loop(0, n_chunks, chunk_body, 0)

    return gather_kernel


def kernel(x, embed_weight):
    n, dim = x.shape
    k, _ = embed_weight.shape
    idx, losssum = _vq_distances(x, embed_weight)
    vq_out = _make_sc_gather(k, dim, n)(embed_weight, idx)
    loss = losssum[0, 0] * (1.25 / (n * dim))
    return (vq_out, loss)


# speedup vs baseline: 1.1674x; 1.0086x over previous
import functools

import jax
import jax.numpy as jnp
from jax import lax
from jax.experimental import pallas as pl
from jax.experimental.pallas import tpu as pltpu
from jax.experimental.pallas import tpu_sc as plsc

BN = 256


def _vq_dist_body(x_ref, w2_ref, x2_ref, z2_ref, idx_ref, losssum_ref):
    nt = pl.program_id(0)
    x = x_ref[...]
    w2 = w2_ref[...]
    k = w2.shape[0]
    s2 = lax.dot_general(x, w2, (((1,), (1,)), ((), ())),
                         preferred_element_type=jnp.float32)
    d = (x2_ref[...] + z2_ref[...]) - s2
    dmin = jnp.min(d, axis=1, keepdims=True)
    iota = lax.broadcasted_iota(jnp.int32, d.shape, 1).astype(jnp.float32)
    idxf = jnp.min(jnp.where(d == dmin, iota, float(k)), axis=1)
    idx_ref[...] = idxf.astype(jnp.int32)

    @pl.when(nt == 0)
    def _():
        losssum_ref[0, 0] = 0.0

    losssum_ref[0, 0] += jnp.sum(dmin)


def _vq_distances(x, embed_weight):
    n, d = x.shape
    k, _ = embed_weight.shape
    x2 = jnp.sum(x ** 2, axis=1, keepdims=True)
    z2 = jnp.sum(embed_weight ** 2, axis=1)[None, :]
    w2 = embed_weight + embed_weight
    grid = (n // BN,)
    return pl.pallas_call(
        _vq_dist_body,
        grid=grid,
        in_specs=[
            pl.BlockSpec((BN, d), lambda i: (i, 0)),
            pl.BlockSpec((k, d), lambda i: (0, 0)),
            pl.BlockSpec((BN, 1), lambda i: (i, 0)),
            pl.BlockSpec((1, k), lambda i: (0, 0)),
        ],
        out_specs=[
            pl.BlockSpec((BN,), lambda i: (i,)),
            pl.BlockSpec(memory_space=pltpu.SMEM),
        ],
        out_shape=[
            jax.ShapeDtypeStruct((n,), jnp.int32),
            jax.ShapeDtypeStruct((1, 1), jnp.float32),
        ],
    )(x, w2, x2, z2)


def _make_sc_gather(v, d, b):
    info = plsc.get_sparse_core_info()
    nw = info.num_cores * info.num_subcores
    b_per_w = b // nw
    chunk = 64
    n_chunks = b_per_w // chunk
    mesh = plsc.VectorSubcoreMesh(core_axis_name="c", subcore_axis_name="s")

    @functools.partial(
        pl.kernel,
        mesh=mesh,
        out_type=jax.ShapeDtypeStruct((b, d), jnp.float32),
        scratch_types=[
            pltpu.VMEM((b_per_w,), jnp.int32),
            pltpu.VMEM((chunk, d), jnp.float32),
            pltpu.SemaphoreType.DMA,
        ],
    )
    def gather_kernel(table_hbm, idx_hbm, out_hbm, idx_v, rows_v, sem):
        wid = lax.axis_index("s") * info.num_cores + lax.axis_index("c")
        base = wid * b_per_w
        pltpu.sync_copy(idx_hbm.at[pl.ds(base, b_per_w)], idx_v)

        def chunk_body(c, carry):
            pltpu.async_copy(
                table_hbm.at[idx_v.at[pl.ds(c * chunk, chunk)]], rows_v, sem
            ).wait()
            pltpu.sync_copy(rows_v, out_hbm.at[pl.ds(base + c * chunk, chunk)])
            return carry

        lax.fori_loop(0, n_chunks, chunk_body, 0)

    return gather_kernel


def kernel(x, embed_weight):
    n, dim = x.shape
    k, _ = embed_weight.shape
    idx, losssum = _vq_distances(x, embed_weight)
    vq_out = _make_sc_gather(k, dim, n)(embed_weight, idx)
    loss = losssum[0, 0] * (1.25 / (n * dim))
    return (vq_out, loss)


# speedup vs baseline: 1.3208x; 1.1314x over previous
import functools

import jax
import jax.numpy as jnp
from jax import lax
from jax.experimental import pallas as pl
from jax.experimental.pallas import tpu as pltpu
from jax.experimental.pallas import tpu_sc as plsc

BN = 256
KC = 256


def _vq_dist_body(x_ref, w2_ref, x2_ref, z2_ref, idx_ref, losssum_ref):
    nt = pl.program_id(0)
    x = x_ref[...]
    w2 = w2_ref[...]
    k = w2.shape[0]
    x2 = x2_ref[...]
    s2 = lax.dot_general(x, w2, (((1,), (1,)), ((), ())),
                         preferred_element_type=jnp.float32)
    n_chunks = k // KC
    mval = None
    mchunk = None
    for c in range(n_chunks):
        z2c = z2_ref[0, c * KC:(c + 1) * KC][None, :]
        dc = (x2 + z2c) - s2[:, c * KC:(c + 1) * KC]
        if c == 0:
            mval = dc
            mchunk = jnp.zeros(dc.shape, jnp.float32)
        else:
            take = dc < mval
            mval = jnp.minimum(mval, dc)
            mchunk = jnp.where(take, float(c), mchunk)
    dmin = jnp.min(mval, axis=1, keepdims=True)
    lane = lax.broadcasted_iota(jnp.int32, mval.shape, 1).astype(jnp.float32)
    cand = mchunk * float(KC) + lane
    idxf = jnp.min(jnp.where(mval == dmin, cand, float(k)), axis=1)
    idx_ref[...] = idxf.astype(jnp.int32)

    @pl.when(nt == 0)
    def _():
        losssum_ref[0, 0] = 0.0

    losssum_ref[0, 0] += jnp.sum(dmin)


def _vq_distances(x, embed_weight):
    n, d = x.shape
    k, _ = embed_weight.shape
    x2 = jnp.sum(x ** 2, axis=1, keepdims=True)
    z2 = jnp.sum(embed_weight ** 2, axis=1)[None, :]
    w2 = embed_weight + embed_weight
    grid = (n // BN,)
    return pl.pallas_call(
        _vq_dist_body,
        grid=grid,
        in_specs=[
            pl.BlockSpec((BN, d), lambda i: (i, 0)),
            pl.BlockSpec((k, d), lambda i: (0, 0)),
            pl.BlockSpec((BN, 1), lambda i: (i, 0)),
            pl.BlockSpec((1, k), lambda i: (0, 0)),
        ],
        out_specs=[
            pl.BlockSpec((BN,), lambda i: (i,)),
            pl.BlockSpec(memory_space=pltpu.SMEM),
        ],
        out_shape=[
            jax.ShapeDtypeStruct((n,), jnp.int32),
            jax.ShapeDtypeStruct((1, 1), jnp.float32),
        ],
    )(x, w2, x2, z2)


def _make_sc_gather(v, d, b):
    info = plsc.get_sparse_core_info()
    nw = info.num_cores * info.num_subcores
    b_per_w = b // nw
    chunk = 64
    n_chunks = b_per_w // chunk
    mesh = plsc.VectorSubcoreMesh(core_axis_name="c", subcore_axis_name="s")

    @functools.partial(
        pl.kernel,
        mesh=mesh,
        out_type=jax.ShapeDtypeStruct((b, d), jnp.float32),
        scratch_types=[
            pltpu.VMEM((b_per_w,), jnp.int32),
            pltpu.VMEM((chunk, d), jnp.float32),
            pltpu.VMEM((chunk, d), jnp.float32),
            pltpu.SemaphoreType.DMA,
            pltpu.SemaphoreType.DMA,
            pltpu.SemaphoreType.DMA,
            pltpu.SemaphoreType.DMA,
        ],
    )
    def gather_kernel(table_hbm, idx_hbm, out_hbm,
                      idx_v, rows0, rows1, gsem0, gsem1, wsem0, wsem1):
        wid = lax.axis_index("s") * info.num_cores + lax.axis_index("c")
        base = wid * b_per_w
        pltpu.sync_copy(idx_hbm.at[pl.ds(base, b_per_w)], idx_v)

        bufs = (rows0, rows1)
        gsems = (gsem0, gsem1)
        wsems = (wsem0, wsem1)

        def gath(c):
            return pltpu.async_copy(
                table_hbm.at[idx_v.at[pl.ds(c * chunk, chunk)]],
                bufs[c % 2], gsems[c % 2])

        writes = [None, None]
        gathers = [None, None]
        gathers[0] = gath(0)
        for c in range(n_chunks):
            nxt = c + 1
            if nxt < n_chunks:
                if writes[nxt % 2] is not None:
                    writes[nxt % 2].wait()
                gathers[nxt % 2] = gath(nxt)
            gathers[c % 2].wait()
            writes[c % 2] = pltpu.async_copy(
                bufs[c % 2], out_hbm.at[pl.ds(base + c * chunk, chunk)],
                wsems[c % 2])
        writes[(n_chunks - 1) % 2].wait()
        if n_chunks >= 2:
            writes[(n_chunks - 2) % 2].wait()

    return gather_kernel


def kernel(x, embed_weight):
    n, dim = x.shape
    k, _ = embed_weight.shape
    idx, losssum = _vq_distances(x, embed_weight)
    vq_out = _make_sc_gather(k, dim, n)(embed_weight, idx)
    loss = losssum[0, 0] * (1.25 / (n * dim))
    return (vq_out, loss)
